# baseline (speedup 1.0000x reference)
import functools

import jax
import jax.numpy as jnp
from jax import lax
from jax.experimental import pallas as pl
from jax.experimental.pallas import tpu as pltpu
from jax.experimental.pallas import tpu_sc as plsc

N = 1024
IN_C = 128
HID = 64
NUM_CLS = 4
E = 16384
PAD = N * (N + 1) // 2 + N + 128
RPG = 64
NC = 2
NS = 16
NW = NC * NS
EPW = E // NW
ROWS_PW = EPW // 128

def _zero16():
    return jnp.zeros((16,), jnp.float32)


def _deg_body(dst_hbm, deg_hbm, idx_v, hist, local, idxw, shared):
    c = lax.axis_index("c")
    s = lax.axis_index("s")
    w = s * NC + c
    pltpu.sync_copy(dst_hbm.at[pl.ds(w * ROWS_PW, ROWS_PW)], idx_v)

    def zrow(k, _):
        for l in range(16):
            hist[l, pl.ds(k * 16, 16)] = _zero16()
        return 0
    lax.fori_loop(0, N // 16, zrow, 0)

    lanes = lax.iota(jnp.int32, 16)
    ones = jnp.ones((16,), jnp.float32)
    for r in range(ROWS_PW):
        for q in range(8):
            idx16 = idx_v[r, pl.ds(q * 16, 16)]
            plsc.addupdate_scatter(hist, [lanes, idx16], ones)

    def red(k, _):
        acc = hist[0, pl.ds(k * 16, 16)]
        for l in range(1, 16):
            acc = acc + hist[l, pl.ds(k * 16, 16)]
        local[k, :] = acc
        return 0
    lax.fori_loop(0, N // 16, red, 0)

    for q in range(4):
        idxw[pl.ds(q * 16, 16)] = lax.iota(jnp.int32, 16) + (q * 16)

    @pl.when(s == 0)
    def _():
        pltpu.sync_copy(local, shared)
    plsc.subcore_barrier()

    @pl.when(s != 0)
    def _():
        pltpu.sync_copy(local, shared.at[idxw], add=True)
    plsc.subcore_barrier()

    @pl.when(s == 0)
    def _():
        pltpu.sync_copy(shared, deg_hbm.at[c])


@functools.cache
def _sc_kernels():
    mesh = plsc.VectorSubcoreMesh(core_axis_name="c", subcore_axis_name="s",
                                  num_cores=NC)
    sc_params = pltpu.CompilerParams(use_tc_tiling_on_sc=False,
                                     needs_layout_passes=False)
    deg_k = pl.kernel(
        _deg_body,
        out_type=jax.ShapeDtypeStruct((NC, N // 16, 16), jnp.float32),
        mesh=mesh,
        scratch_types=[
            pltpu.VMEM((ROWS_PW, 128), jnp.int32),
            pltpu.VMEM((16, N), jnp.float32),
            pltpu.VMEM((N // 16, 16), jnp.float32),
            pltpu.VMEM((N // 16,), jnp.int32),
            pltpu.VMEM_SHARED((N // 16, 16), jnp.float32),
        ],
        compiler_params=sc_params,
    )
    agg_k = pl.kernel(
        _agg_body,
        out_type=jax.ShapeDtypeStruct((NC, N, HID), jnp.float32),
        mesh=mesh,
        scratch_types=[
            pltpu.VMEM((ROWS_PW, 128), jnp.int32),
            pltpu.VMEM((ROWS_PW, 128), jnp.int32),
            pltpu.VMEM((ROWS_PW, 128, HID), jnp.float32),
            pltpu.VMEM((N // NS, HID), jnp.float32),
            pltpu.VMEM_SHARED((N, HID), jnp.float32),
            pltpu.SemaphoreType.DMA,
        ],
        compiler_params=sc_params,
    )
    return deg_k, agg_k


def _agg_body(y_hbm, src_hbm, dst_hbm, out_hbm, srcv, dstv, rows, zb, shared,
              sem):
    c = lax.axis_index("c")
    s = lax.axis_index("s")
    w = s * NC + c
    pltpu.sync_copy(src_hbm.at[pl.ds(w * ROWS_PW, ROWS_PW)], srcv)
    pltpu.sync_copy(dst_hbm.at[pl.ds(w * ROWS_PW, ROWS_PW)], dstv)

    descs = [pltpu.async_copy(y_hbm.at[srcv.at[j]], rows.at[j], sem)
             for j in range(ROWS_PW)]

    for r in range(N // NS):
        for q in range(HID // 16):
            zb[r, pl.ds(q * 16, 16)] = _zero16()
    pltpu.sync_copy(zb, shared.at[pl.ds(s * (N // NS), N // NS)])
    plsc.subcore_barrier()

    for d in descs:
        d.wait()
    for j in range(ROWS_PW):
        pltpu.sync_copy(rows.at[j], shared.at[dstv.at[j]], add=True)
    plsc.subcore_barrier()

    sl = pl.ds(s * (N // NS), N // NS)
    pltpu.sync_copy(shared.at[sl], out_hbm.at[c].at[sl])


def _prep_kernel(x_ref, W1_ref, deg_ref, y1_ref, dinv_ref):
    xw = jnp.dot(x_ref[...], W1_ref[...], preferred_element_type=jnp.float32)
    dinv = lax.rsqrt(deg_ref[0] + deg_ref[1] + 1.0)
    dinv_ref[...] = dinv
    y1_ref[...] = xw * dinv


def _mid_kernel(agg_ref, y1_ref, dinv_ref, b1_ref, W2_ref, y2_ref):
    dinv = dinv_ref[...]
    h1 = jnp.maximum((agg_ref[0] + agg_ref[1] + y1_ref[...]) * dinv
                     + b1_ref[...], 0.0)
    y2_ref[...] = jnp.dot(h1, W2_ref[...],
                          preferred_element_type=jnp.float32) * dinv


def _pair_kernel(agg_ref, y2_ref, dinv_ref, b2_ref,
                 mW1aT_ref, mW1bT_ref, mb1_ref, mW2T_ref, mb2_ref,
                 logits_ref, pairs_ref, h_ref, a_s, b_s):
    i = pl.program_id(0)

    @pl.when(i == 0)
    def _():
        h = jnp.maximum(
            (agg_ref[0] + agg_ref[1] + y2_ref[...]) * dinv_ref[...]
            + b2_ref[...], 0.0)
        h_ref[...] = h
        ht = h.T
        at = jnp.dot(mW1aT_ref[...], ht,
                     preferred_element_type=jnp.float32) + mb1_ref[...]
        a_s[...] = at.astype(jnp.bfloat16)
        bt = jnp.dot(mW1bT_ref[...], ht, preferred_element_type=jnp.float32)
        btb = bt.astype(jnp.bfloat16)
        b_s[:, 0:N] = btb
        b_s[:, N:2 * N] = btb

    i0 = i * RPG
    ib = pl.multiple_of((i0 // 128) * 128, 128)
    ri0 = i0 - ib
    arot = pltpu.roll(a_s[:, pl.ds(ib, 128)], 128 - ri0, axis=1)

    def tier(WT):
        WS = WT + 128
        slab = b_s[:, pl.ds(ib, WT)]
        lane4 = lax.broadcasted_iota(jnp.int32, (NUM_CLS, WS), 1)
        lane2 = lax.broadcasted_iota(jnp.int32, (2, WS), 1)
        jrow = ib + lax.broadcasted_iota(jnp.int32, (1, WT), 1)
        for r in range(RPG):
            iw = i0 + r
            ri = ri0 + r
            acol = arot[:, r:r + 1]
            hid = jnp.maximum(acol + slab, jnp.bfloat16(0.0))
            lg = jnp.dot(mW2T_ref[...], hid,
                         preferred_element_type=jnp.float32) + mb2_ref[...]

            off = (iw * (2 * N + 1 - iw)) // 2
            t0 = off - ri
            tb = pl.multiple_of((t0 // 128) * 128, 128)
            rt = t0 - tb
            lgw = jnp.concatenate([lg, lg[:, :128]], axis=1)
            lgw = pltpu.roll(lgw, rt, axis=1)
            old = logits_ref[:, pl.ds(tb, WS)]
            logits_ref[:, pl.ds(tb, WS)] = jnp.where(lane4 >= rt + ri,
                                                     lgw, old)

            pr = jnp.concatenate(
                [jnp.full((1, WT), iw, jnp.int32), jrow], axis=0)
            prw = jnp.concatenate([pr, pr[:, :128]], axis=1)
            prw = pltpu.roll(prw, rt, axis=1)
            oldp = pairs_ref[:, pl.ds(tb, WS)]
            pairs_ref[:, pl.ds(tb, WS)] = jnp.where(lane2 >= rt + ri,
                                                    prw, oldp)

    nt = N // 256
    for t in range(nt):
        @pl.when(i0 // 256 == t)
        def _(t=t):
            tier(N + 128 - 256 * t)


def _full(shape):
    return pl.BlockSpec(shape, lambda *a: tuple(0 for _ in shape))


def _pair_stage(agg2, y2, dinv, b2, mW1, mb1, mW2, mb2):
    mW1aT = mW1[:HID].T
    mW1bT = mW1[HID:].T
    mb1c = mb1[:, None]
    mW2T = mW2.T.astype(jnp.bfloat16)
    mb2c = mb2[:, None]
    b2r = b2[None, :]
    logits_t, pairs_t, h = pl.pallas_call(
        _pair_kernel,
        grid=(N // RPG,),
        in_specs=[_full(s.shape) for s in
                  (agg2, y2, dinv, b2r, mW1aT, mW1bT, mb1c, mW2T, mb2c)],
        out_specs=[
            _full((NUM_CLS, PAD)),
            _full((2, PAD)),
            _full((N, HID)),
        ],
        out_shape=[
            jax.ShapeDtypeStruct((NUM_CLS, PAD), jnp.float32),
            jax.ShapeDtypeStruct((2, PAD), jnp.int32),
            jax.ShapeDtypeStruct((N, HID), jnp.float32),
        ],
        scratch_shapes=[
            pltpu.VMEM((HID, N), jnp.bfloat16),
            pltpu.VMEM((HID, 2 * N), jnp.bfloat16),
        ],
    )(agg2, y2, dinv, b2r, mW1aT, mW1bT, mb1c, mW2T, mb2c)
    npairs = N * (N + 1) // 2
    return logits_t[:, :npairs].T, pairs_t[:, :npairs].T, h


def kernel(x, edge_index, W1, b1, W2, b2, mW1, mb1, mW2, mb2):
    src2d = edge_index[0].astype(jnp.int32).reshape(E // 128, 128)
    dst2d = edge_index[1].astype(jnp.int32).reshape(E // 128, 128)

    deg_k, agg_k = _sc_kernels()
    deg2 = deg_k(dst2d).reshape(NC, N, 1)

    y1, dinv = pl.pallas_call(
        _prep_kernel,
        in_specs=[_full(x.shape), _full(W1.shape), _full((NC, N, 1))],
        out_specs=[_full((N, HID)), _full((N, 1))],
        out_shape=[jax.ShapeDtypeStruct((N, HID), jnp.float32),
                   jax.ShapeDtypeStruct((N, 1), jnp.float32)],
    )(x, W1, deg2)

    agg1 = agg_k(y1, src2d, dst2d)

    y2 = pl.pallas_call(
        _mid_kernel,
        in_specs=[_full((NC, N, HID)), _full((N, HID)), _full((N, 1)),
                  _full((1, HID)), _full(W2.shape)],
        out_specs=_full((N, HID)),
        out_shape=jax.ShapeDtypeStruct((N, HID), jnp.float32),
    )(agg1, y1, dinv, b1[None, :], W2)

    agg2 = agg_k(y2, src2d, dst2d)

    logits, pairs, h = _pair_stage(agg2, y2, dinv, b2, mW1, mb1, mW2, mb2)
    return (logits, pairs.astype(jnp.int64), h)


# speedup vs baseline: 1.1090x; 1.1090x over previous
import functools

import jax
import jax.numpy as jnp
from jax import lax
from jax.experimental import pallas as pl
from jax.experimental.pallas import tpu as pltpu
from jax.experimental.pallas import tpu_sc as plsc

N = 1024
IN_C = 128
HID = 64
NUM_CLS = 4
E = 16384
PAD = N * (N + 1) // 2 + N + 128
RPG = 32
NC = 2
NS = 16
NW = NC * NS
EPW = E // NW
ROWS_PW = EPW // 128

def _zero16():
    return jnp.zeros((16,), jnp.float32)


def _deg_body(dst_hbm, deg_hbm, idx_v, hist, local, idxw, shared):
    c = lax.axis_index("c")
    s = lax.axis_index("s")
    w = s * NC + c
    pltpu.sync_copy(dst_hbm.at[pl.ds(w * ROWS_PW, ROWS_PW)], idx_v)

    def zrow(k, _):
        for l in range(16):
            hist[l, pl.ds(k * 16, 16)] = _zero16()
        return 0
    lax.fori_loop(0, N // 16, zrow, 0)

    lanes = lax.iota(jnp.int32, 16)
    ones = jnp.ones((16,), jnp.float32)
    for r in range(ROWS_PW):
        for q in range(8):
            idx16 = idx_v[r, pl.ds(q * 16, 16)]
            plsc.addupdate_scatter(hist, [lanes, idx16], ones)

    def red(k, _):
        acc = hist[0, pl.ds(k * 16, 16)]
        for l in range(1, 16):
            acc = acc + hist[l, pl.ds(k * 16, 16)]
        local[k, :] = acc
        return 0
    lax.fori_loop(0, N // 16, red, 0)

    for q in range(4):
        idxw[pl.ds(q * 16, 16)] = lax.iota(jnp.int32, 16) + (q * 16)

    @pl.when(s == 0)
    def _():
        pltpu.sync_copy(local, shared)
    plsc.subcore_barrier()

    @pl.when(s != 0)
    def _():
        pltpu.sync_copy(local, shared.at[idxw], add=True)
    plsc.subcore_barrier()

    @pl.when(s == 0)
    def _():
        pltpu.sync_copy(shared, deg_hbm.at[c])


@functools.cache
def _sc_kernels():
    mesh = plsc.VectorSubcoreMesh(core_axis_name="c", subcore_axis_name="s",
                                  num_cores=NC)
    sc_params = pltpu.CompilerParams(use_tc_tiling_on_sc=False,
                                     needs_layout_passes=False)
    deg_k = pl.kernel(
        _deg_body,
        out_type=jax.ShapeDtypeStruct((NC, N // 16, 16), jnp.float32),
        mesh=mesh,
        scratch_types=[
            pltpu.VMEM((ROWS_PW, 128), jnp.int32),
            pltpu.VMEM((16, N), jnp.float32),
            pltpu.VMEM((N // 16, 16), jnp.float32),
            pltpu.VMEM((N // 16,), jnp.int32),
            pltpu.VMEM_SHARED((N // 16, 16), jnp.float32),
        ],
        compiler_params=sc_params,
    )
    agg_k = pl.kernel(
        _agg_body,
        out_type=jax.ShapeDtypeStruct((NC, N, HID), jnp.float32),
        mesh=mesh,
        scratch_types=[
            pltpu.VMEM((ROWS_PW, 128), jnp.int32),
            pltpu.VMEM((ROWS_PW, 128), jnp.int32),
            pltpu.VMEM((ROWS_PW, 128, HID), jnp.float32),
            pltpu.VMEM((N // NS, HID), jnp.float32),
            pltpu.VMEM_SHARED((N, HID), jnp.float32),
            pltpu.SemaphoreType.DMA,
        ],
        compiler_params=sc_params,
    )
    return deg_k, agg_k


def _agg_body(y_hbm, src_hbm, dst_hbm, out_hbm, srcv, dstv, rows, zb, shared,
              sem):
    c = lax.axis_index("c")
    s = lax.axis_index("s")
    w = s * NC + c
    pltpu.sync_copy(src_hbm.at[pl.ds(w * ROWS_PW, ROWS_PW)], srcv)
    pltpu.sync_copy(dst_hbm.at[pl.ds(w * ROWS_PW, ROWS_PW)], dstv)

    descs = [pltpu.async_copy(y_hbm.at[srcv.at[j]], rows.at[j], sem)
             for j in range(ROWS_PW)]

    for r in range(N // NS):
        for q in range(HID // 16):
            zb[r, pl.ds(q * 16, 16)] = _zero16()
    pltpu.sync_copy(zb, shared.at[pl.ds(s * (N // NS), N // NS)])
    plsc.subcore_barrier()

    for d in descs:
        d.wait()
    for j in range(ROWS_PW):
        pltpu.sync_copy(rows.at[j], shared.at[dstv.at[j]], add=True)
    plsc.subcore_barrier()

    sl = pl.ds(s * (N // NS), N // NS)
    pltpu.sync_copy(shared.at[sl], out_hbm.at[c].at[sl])


def _prep_kernel(x_ref, W1_ref, deg_ref, y1_ref, dinv_ref):
    xw = jnp.dot(x_ref[...], W1_ref[...], preferred_element_type=jnp.float32)
    dinv = lax.rsqrt(deg_ref[0] + deg_ref[1] + 1.0)
    dinv_ref[...] = dinv
    y1_ref[...] = xw * dinv


def _mid_kernel(agg_ref, y1_ref, dinv_ref, b1_ref, W2_ref, y2_ref):
    dinv = dinv_ref[...]
    h1 = jnp.maximum((agg_ref[0] + agg_ref[1] + y1_ref[...]) * dinv
                     + b1_ref[...], 0.0)
    y2_ref[...] = jnp.dot(h1, W2_ref[...],
                          preferred_element_type=jnp.float32) * dinv


def _pair_kernel(agg_ref, y2_ref, dinv_ref, b2_ref,
                 mW1aT_ref, mW1bT_ref, mb1_ref, mW2T_ref, mb2_ref,
                 logits_ref, pairs_ref, h_ref, a_s, b_s):
    i = pl.program_id(0)

    @pl.when(i == 0)
    def _():
        h = jnp.maximum(
            (agg_ref[0] + agg_ref[1] + y2_ref[...]) * dinv_ref[...]
            + b2_ref[...], 0.0)
        h_ref[...] = h
        ht = h.T
        at = jnp.dot(mW1aT_ref[...], ht,
                     preferred_element_type=jnp.float32) + mb1_ref[...]
        a_s[...] = at.astype(jnp.bfloat16)
        bt = jnp.dot(mW1bT_ref[...], ht, preferred_element_type=jnp.float32)
        btb = bt.astype(jnp.bfloat16)
        b_s[:, 0:N] = btb
        b_s[:, N:2 * N] = btb

    i0 = i * RPG
    ib = pl.multiple_of((i0 // 128) * 128, 128)
    ri0 = i0 - ib
    arot = pltpu.roll(a_s[:, pl.ds(ib, 128)], 128 - ri0, axis=1)

    def tier(WT):
        WS = WT + 128
        slab = b_s[:, pl.ds(ib, WT)]
        lane4 = lax.broadcasted_iota(jnp.int32, (NUM_CLS, WS), 1)
        lane2 = lax.broadcasted_iota(jnp.int32, (2, WS), 1)
        jrow = ib + lax.broadcasted_iota(jnp.int32, (1, WT), 1)
        for r in range(RPG):
            iw = i0 + r
            ri = ri0 + r
            acol = arot[:, r:r + 1]
            hid = jnp.maximum(acol + slab, jnp.bfloat16(0.0))
            lg = jnp.dot(mW2T_ref[...], hid,
                         preferred_element_type=jnp.float32) + mb2_ref[...]

            off = (iw * (2 * N + 1 - iw)) // 2
            t0 = off - ri
            tb = pl.multiple_of((t0 // 128) * 128, 128)
            rt = t0 - tb
            lgw = jnp.concatenate([lg, lg[:, :128]], axis=1)
            lgw = pltpu.roll(lgw, rt, axis=1)
            old = logits_ref[:, pl.ds(tb, WS)]
            logits_ref[:, pl.ds(tb, WS)] = jnp.where(lane4 >= rt + ri,
                                                     lgw, old)

            pr = jnp.concatenate(
                [jnp.full((1, WT), iw, jnp.int32), jrow], axis=0)
            prw = jnp.concatenate([pr, pr[:, :128]], axis=1)
            prw = pltpu.roll(prw, rt, axis=1)
            oldp = pairs_ref[:, pl.ds(tb, WS)]
            pairs_ref[:, pl.ds(tb, WS)] = jnp.where(lane2 >= rt + ri,
                                                    prw, oldp)

    nt = N // 256
    for t in range(nt):
        @pl.when(i0 // 256 == t)
        def _(t=t):
            tier(N + 128 - 256 * t)


def _full(shape):
    return pl.BlockSpec(shape, lambda *a: tuple(0 for _ in shape))


def _pair_stage(agg2, y2, dinv, b2, mW1, mb1, mW2, mb2):
    mW1aT = mW1[:HID].T
    mW1bT = mW1[HID:].T
    mb1c = mb1[:, None]
    mW2T = mW2.T.astype(jnp.bfloat16)
    mb2c = mb2[:, None]
    b2r = b2[None, :]
    logits_t, pairs_t, h = pl.pallas_call(
        _pair_kernel,
        grid=(N // RPG,),
        in_specs=[_full(s.shape) for s in
                  (agg2, y2, dinv, b2r, mW1aT, mW1bT, mb1c, mW2T, mb2c)],
        out_specs=[
            _full((NUM_CLS, PAD)),
            _full((2, PAD)),
            _full((N, HID)),
        ],
        out_shape=[
            jax.ShapeDtypeStruct((NUM_CLS, PAD), jnp.float32),
            jax.ShapeDtypeStruct((2, PAD), jnp.int32),
            jax.ShapeDtypeStruct((N, HID), jnp.float32),
        ],
        scratch_shapes=[
            pltpu.VMEM((HID, N), jnp.bfloat16),
            pltpu.VMEM((HID, 2 * N), jnp.bfloat16),
        ],
    )(agg2, y2, dinv, b2r, mW1aT, mW1bT, mb1c, mW2T, mb2c)
    npairs = N * (N + 1) // 2
    return logits_t[:, :npairs].T, pairs_t[:, :npairs].T, h


def kernel(x, edge_index, W1, b1, W2, b2, mW1, mb1, mW2, mb2):
    src2d = edge_index[0].astype(jnp.int32).reshape(E // 128, 128)
    dst2d = edge_index[1].astype(jnp.int32).reshape(E // 128, 128)

    deg_k, agg_k = _sc_kernels()
    deg2 = deg_k(dst2d).reshape(NC, N, 1)

    y1, dinv = pl.pallas_call(
        _prep_kernel,
        in_specs=[_full(x.shape), _full(W1.shape), _full((NC, N, 1))],
        out_specs=[_full((N, HID)), _full((N, 1))],
        out_shape=[jax.ShapeDtypeStruct((N, HID), jnp.float32),
                   jax.ShapeDtypeStruct((N, 1), jnp.float32)],
    )(x, W1, deg2)

    agg1 = agg_k(y1, src2d, dst2d)

    y2 = pl.pallas_call(
        _mid_kernel,
        in_specs=[_full((NC, N, HID)), _full((N, HID)), _full((N, 1)),
                  _full((1, HID)), _full(W2.shape)],
        out_specs=_full((N, HID)),
        out_shape=jax.ShapeDtypeStruct((N, HID), jnp.float32),
    )(agg1, y1, dinv, b1[None, :], W2)

    agg2 = agg_k(y2, src2d, dst2d)

    logits, pairs, h = _pair_stage(agg2, y2, dinv, b2, mW1, mb1, mW2, mb2)
    return (logits, pairs.astype(jnp.int64), h)
